# baseline (speedup 1.0000x reference)
import functools

import jax
import jax.numpy as jnp
from jax import lax
from jax.experimental import pallas as pl
from jax.experimental.pallas import tpu as pltpu, tpu_sc as plsc

_N = 4096
_D = 512
_BLK = 256
_NC = 2
_NS = 16
_NW = _NC * _NS
_RPW = _N // _NW


def _tc_body(o1_ref, o2t_ref, keys_ref, pos_ref):
    r0 = pl.program_id(0) * _BLK
    o1 = o1_ref[...]
    o2t = o2t_ref[...]
    n1 = jnp.sum(o1 * o1, axis=1, keepdims=True)
    n2 = jnp.sum(o2t * o2t, axis=0, keepdims=True)
    dot = jnp.dot(o1, o2t, preferred_element_type=jnp.float32)
    d2 = n1 - 2.0 * dot + n2
    cols = jax.lax.broadcasted_iota(jnp.int32, (_BLK, _N), 1)
    rows = jax.lax.broadcasted_iota(jnp.int32, (_BLK, _N), 0) + r0
    diag = cols == rows
    d2 = jnp.where(diag, jnp.inf, d2)
    d2 = jnp.maximum(d2, 1e-12)
    keys_ref[...] = jax.lax.bitcast_convert_type(d2, jnp.int32)

    dmask = diag.astype(jnp.float32)
    dd = jnp.sum(dot * dmask, axis=1, keepdims=True)
    n2d = jnp.sum(n2 * dmask, axis=1, keepdims=True)
    pos_ref[...] = n1 + n2d - 2.0 * dd


def _splat(x):
    return jnp.full((16,), x, jnp.int32)


_GDN = jax.lax.GatherDimensionNumbers(
    offset_dims=(), collapsed_slice_dims=(0,), start_index_map=(0,))


def _lane_gather(x, idx_vec):
    return jax.lax.gather(x, idx_vec[:, None], _GDN, (1,),
                          mode=jax.lax.GatherScatterMode.PROMISE_IN_BOUNDS)


def _last_lane(x):
    return _lane_gather(x, jnp.full((16,), 15, jnp.int32))


def _sc_body(keys_hbm, rn_hbm, out_hbm, kbuf, hist, hist2, coarse, coarse2,
             rnv, chosen, sems):
    wid = lax.axis_index("s") * _NC + lax.axis_index("c")
    base = wid * _RPW
    pltpu.sync_copy(rn_hbm.at[pl.ds(base, _RPW)], rnv)

    lanes = lax.iota(jnp.int32, 16)
    zeros16 = jnp.zeros((16,), jnp.int32)
    ones16 = jnp.full((16,), 1, jnp.int32)
    lane0 = lanes == 0

    def _block_copy(rb, par):
        return pltpu.make_async_copy(
            keys_hbm.at[pl.ds(base + rb * 8, 8)], kbuf.at[par], sems.at[par])

    _block_copy(0, 0).start()

    def block_step(rb, _):
        par = rb % 2
        _block_copy(rb, par).wait()

        @pl.when(rb < _RPW // 8 - 1)
        def _start_next():
            _block_copy(rb + 1, 1 - par).start()

        def row_step(rloc, _):
            r = rb * 8 + rloc
            _select_row(r, rloc, par)
            return 0
        lax.fori_loop(0, 8, row_step, 0)
        return 0

    def _select_row(r, rloc, par):
        @plsc.parallel_loop(0, 2048, step=16, unroll=8)
        def _z1(i):
            hist[pl.ds(i, 16)] = zeros16

        @plsc.parallel_loop(0, 1024, step=16, unroll=8)
        def _z2(i):
            hist2[pl.ds(i, 16)] = zeros16

        for t in range(8):
            coarse[pl.ds(t * 16, 16)] = zeros16
        for t in range(4):
            coarse2[pl.ds(t * 16, 16)] = zeros16

        k_vec = plsc.load_gather(rnv, [_splat(r)])

        @plsc.parallel_loop(0, _N, step=16, unroll=8)
        def _p1(i):
            v = kbuf[par, rloc, pl.ds(i, 16)]
            b = (v >> 20) & 2047
            cf, lastf = plsc.scan_count(b)
            plsc.addupdate_scatter(hist, [b], cf, mask=lastf)
            cc, last = plsc.scan_count(b >> 4)
            plsc.addupdate_scatter(coarse, [b >> 4], cc, mask=last)

        x1, rin1 = _scan_find(coarse, 8, k_vec)
        b1, r2 = _fine_find(hist, x1, rin1)

        @plsc.parallel_loop(0, _N, step=16, unroll=8)
        def _p2(i):
            v = kbuf[par, rloc, pl.ds(i, 16)]
            t = v >> 10
            m = (t >> 10) == b1
            b = t & 1023
            cf, lastf = plsc.scan_count(b, mask=m)
            plsc.addupdate_scatter(hist2, [b], cf, mask=lastf & m)
            cc, last = plsc.scan_count(b >> 4, mask=m)
            plsc.addupdate_scatter(coarse2, [b >> 4], cc, mask=last & m)

        x2, rin2 = _scan_find(coarse2, 4, r2)
        b2, _ = _fine_find(hist2, x2, rin2)

        bits = (b1 << 20) | (b2 << 10) | 512
        plsc.store_scatter(chosen, [_splat(r)], bits, mask=lane0)

    def _scan_find(c_ref, nt, rank):
        cs = [plsc.cumsum(c_ref[pl.ds(16 * t, 16)]) for t in range(nt)]
        run = zeros16
        cnt = zeros16
        below_acc = zeros16
        for t in range(nt):
            c = run + cs[t]
            m = c <= rank
            cnt = cnt + plsc.all_reduce_population_count(m)
            below_acc = jnp.maximum(below_acc, jnp.where(m, c, 0))
            if t < nt - 1:
                run = _last_lane(c)
        below = _splat(jnp.max(below_acc))
        return cnt, rank - below

    def _fine_find(h_ref, x, rank):
        h = plsc.load_gather(h_ref, [x * 16 + lanes])
        c = plsc.cumsum(h)
        m = c <= rank
        lane = plsc.all_reduce_population_count(m)
        below = jnp.where(lane == 0, 0,
                          _lane_gather(c, jnp.maximum(lane - 1, 0)))
        return x * 16 + lane, rank - below

    lax.fori_loop(0, _RPW // 8, block_step, 0)
    pltpu.sync_copy(chosen, out_hbm.at[pl.ds(base, _RPW)])


@jax.jit
def _run(output1, output2, rn):
    o2t = output2.T
    keys, pos = pl.pallas_call(
        _tc_body,
        grid=(_N // _BLK,),
        in_specs=[
            pl.BlockSpec((_BLK, _D), lambda i: (i, 0)),
            pl.BlockSpec((_D, _N), lambda i: (0, 0)),
        ],
        out_specs=[
            pl.BlockSpec((_BLK, _N), lambda i: (i, 0)),
            pl.BlockSpec((_BLK, 1), lambda i: (i, 0)),
        ],
        out_shape=[
            jax.ShapeDtypeStruct((_N, _N), jnp.int32),
            jax.ShapeDtypeStruct((_N, 1), jnp.float32),
        ],
    )(output1, o2t)

    sc_select = pl.kernel(
        _sc_body,
        out_type=jax.ShapeDtypeStruct((_N,), jnp.int32),
        mesh=plsc.VectorSubcoreMesh(core_axis_name="c", subcore_axis_name="s"),
        compiler_params=pltpu.CompilerParams(needs_layout_passes=False),
        scratch_types=[
            pltpu.VMEM((2, 8, _N), jnp.int32),
            pltpu.VMEM((2048,), jnp.int32),
            pltpu.VMEM((1024,), jnp.int32),
            pltpu.VMEM((128,), jnp.int32),
            pltpu.VMEM((64,), jnp.int32),
            pltpu.VMEM((_RPW,), jnp.int32),
            pltpu.VMEM((_RPW,), jnp.int32),
            pltpu.SemaphoreType.DMA((2,)),
        ],
    )
    chosen_bits = sc_select(keys, rn)

    dist = jnp.sqrt(jax.lax.bitcast_convert_type(chosen_bits, jnp.float32))
    neg_loss = jnp.clip(2.0 - dist, 0.0, None)
    return jnp.mean(pos[:, 0]) + jnp.mean(neg_loss)


def kernel(output1, output2, quant):
    N = output1.shape[0]
    q = min(100, N - 1)
    rn = jax.random.randint(jax.random.key(1234), (N,), 0, q)
    rn = jnp.minimum(rn, quant - 1).astype(jnp.int32)
    return _run(output1, output2, rn)


# speedup vs baseline: 1.0103x; 1.0103x over previous
import functools

import jax
import jax.numpy as jnp
from jax import lax
from jax.experimental import pallas as pl
from jax.experimental.pallas import tpu as pltpu, tpu_sc as plsc

_N = 4096
_D = 512
_BLK = 256
_NC = 2
_NS = 16
_NW = _NC * _NS
_RPW = _N // _NW


def _tc_body(o1_ref, o2t_ref, keys_ref, pos_ref):
    r0 = pl.program_id(0) * _BLK
    o1 = o1_ref[...]
    o2t = o2t_ref[...]
    n1 = jnp.sum(o1 * o1, axis=1, keepdims=True)
    n2 = jnp.sum(o2t * o2t, axis=0, keepdims=True)
    dot = jnp.dot(o1.astype(jnp.bfloat16), o2t.astype(jnp.bfloat16),
                  preferred_element_type=jnp.float32)
    d2 = n1 - 2.0 * dot + n2
    cols = jax.lax.broadcasted_iota(jnp.int32, (_BLK, _N), 1)
    rows = jax.lax.broadcasted_iota(jnp.int32, (_BLK, _N), 0) + r0
    diag = cols == rows
    d2 = jnp.where(diag, jnp.inf, d2)
    d2 = jnp.maximum(d2, 1e-12)
    keys_ref[...] = jax.lax.bitcast_convert_type(d2, jnp.int32)

    dmask = diag.astype(jnp.float32)
    dd = jnp.sum(dot * dmask, axis=1, keepdims=True)
    n2d = jnp.sum(n2 * dmask, axis=1, keepdims=True)
    pos_ref[...] = n1 + n2d - 2.0 * dd


def _splat(x):
    return jnp.full((16,), x, jnp.int32)


_GDN = jax.lax.GatherDimensionNumbers(
    offset_dims=(), collapsed_slice_dims=(0,), start_index_map=(0,))


def _lane_gather(x, idx_vec):
    return jax.lax.gather(x, idx_vec[:, None], _GDN, (1,),
                          mode=jax.lax.GatherScatterMode.PROMISE_IN_BOUNDS)


def _last_lane(x):
    return _lane_gather(x, jnp.full((16,), 15, jnp.int32))


def _sc_body(keys_hbm, rn_hbm, out_hbm, kbuf, hist, hist2, coarse, coarse2,
             rnv, chosen, sems):
    wid = lax.axis_index("s") * _NC + lax.axis_index("c")
    base = wid * _RPW
    pltpu.sync_copy(rn_hbm.at[pl.ds(base, _RPW)], rnv)

    lanes = lax.iota(jnp.int32, 16)
    zeros16 = jnp.zeros((16,), jnp.int32)
    ones16 = jnp.full((16,), 1, jnp.int32)
    lane0 = lanes == 0

    def _block_copy(rb, par):
        return pltpu.make_async_copy(
            keys_hbm.at[pl.ds(base + rb * 8, 8)], kbuf.at[par], sems.at[par])

    _block_copy(0, 0).start()

    def block_step(rb, _):
        par = rb % 2
        _block_copy(rb, par).wait()

        @pl.when(rb < _RPW // 8 - 1)
        def _start_next():
            _block_copy(rb + 1, 1 - par).start()

        def row_step(rloc, _):
            r = rb * 8 + rloc
            _select_row(r, rloc, par)
            return 0
        lax.fori_loop(0, 8, row_step, 0)
        return 0

    def _select_row(r, rloc, par):
        @plsc.parallel_loop(0, 2048, step=16, unroll=8)
        def _z1(i):
            hist[pl.ds(i, 16)] = zeros16

        @plsc.parallel_loop(0, 1024, step=16, unroll=8)
        def _z2(i):
            hist2[pl.ds(i, 16)] = zeros16

        for t in range(8):
            coarse[pl.ds(t * 16, 16)] = zeros16
        for t in range(4):
            coarse2[pl.ds(t * 16, 16)] = zeros16

        k_vec = plsc.load_gather(rnv, [_splat(r)])

        @plsc.parallel_loop(0, _N, step=16, unroll=8)
        def _p1(i):
            v = kbuf[par, rloc, pl.ds(i, 16)]
            b = (v >> 20) & 2047
            plsc.addupdate_scatter(hist, [b], ones16)
            cc, last = plsc.scan_count(b >> 4)
            plsc.addupdate_scatter(coarse, [b >> 4], cc, mask=last)

        x1, rin1 = _scan_find(coarse, 8, k_vec)
        b1, r2 = _fine_find(hist, x1, rin1)

        @plsc.parallel_loop(0, _N, step=16, unroll=8)
        def _p2(i):
            v = kbuf[par, rloc, pl.ds(i, 16)]
            t = v >> 10
            m = (t >> 10) == b1
            b = t & 1023
            plsc.addupdate_scatter(hist2, [b], ones16, mask=m)
            cc, last = plsc.scan_count(b >> 4, mask=m)
            plsc.addupdate_scatter(coarse2, [b >> 4], cc, mask=last & m)

        x2, rin2 = _scan_find(coarse2, 4, r2)
        b2, _ = _fine_find(hist2, x2, rin2)

        bits = (b1 << 20) | (b2 << 10) | 512
        plsc.store_scatter(chosen, [_splat(r)], bits, mask=lane0)

    def _scan_find(c_ref, nt, rank):
        cs = [plsc.cumsum(c_ref[pl.ds(16 * t, 16)]) for t in range(nt)]
        run = zeros16
        cnt = zeros16
        below_acc = zeros16
        for t in range(nt):
            c = run + cs[t]
            m = c <= rank
            cnt = cnt + plsc.all_reduce_population_count(m)
            below_acc = jnp.maximum(below_acc, jnp.where(m, c, 0))
            if t < nt - 1:
                run = _last_lane(c)
        below = _splat(jnp.max(below_acc))
        return cnt, rank - below

    def _fine_find(h_ref, x, rank):
        h = plsc.load_gather(h_ref, [x * 16 + lanes])
        c = plsc.cumsum(h)
        m = c <= rank
        lane = plsc.all_reduce_population_count(m)
        below = jnp.where(lane == 0, 0,
                          _lane_gather(c, jnp.maximum(lane - 1, 0)))
        return x * 16 + lane, rank - below

    lax.fori_loop(0, _RPW // 8, block_step, 0)
    pltpu.sync_copy(chosen, out_hbm.at[pl.ds(base, _RPW)])


@jax.jit
def _run(output1, output2, rn):
    o2t = output2.T
    keys, pos = pl.pallas_call(
        _tc_body,
        grid=(_N // _BLK,),
        in_specs=[
            pl.BlockSpec((_BLK, _D), lambda i: (i, 0)),
            pl.BlockSpec((_D, _N), lambda i: (0, 0)),
        ],
        out_specs=[
            pl.BlockSpec((_BLK, _N), lambda i: (i, 0)),
            pl.BlockSpec((_BLK, 1), lambda i: (i, 0)),
        ],
        out_shape=[
            jax.ShapeDtypeStruct((_N, _N), jnp.int32),
            jax.ShapeDtypeStruct((_N, 1), jnp.float32),
        ],
    )(output1, o2t)

    sc_select = pl.kernel(
        _sc_body,
        out_type=jax.ShapeDtypeStruct((_N,), jnp.int32),
        mesh=plsc.VectorSubcoreMesh(core_axis_name="c", subcore_axis_name="s"),
        compiler_params=pltpu.CompilerParams(needs_layout_passes=False),
        scratch_types=[
            pltpu.VMEM((2, 8, _N), jnp.int32),
            pltpu.VMEM((2048,), jnp.int32),
            pltpu.VMEM((1024,), jnp.int32),
            pltpu.VMEM((128,), jnp.int32),
            pltpu.VMEM((64,), jnp.int32),
            pltpu.VMEM((_RPW,), jnp.int32),
            pltpu.VMEM((_RPW,), jnp.int32),
            pltpu.SemaphoreType.DMA((2,)),
        ],
    )
    chosen_bits = sc_select(keys, rn)

    dist = jnp.sqrt(jax.lax.bitcast_convert_type(chosen_bits, jnp.float32))
    neg_loss = jnp.clip(2.0 - dist, 0.0, None)
    return jnp.mean(pos[:, 0]) + jnp.mean(neg_loss)


def kernel(output1, output2, quant):
    N = output1.shape[0]
    q = min(100, N - 1)
    rn = jax.random.randint(jax.random.key(1234), (N,), 0, q)
    rn = jnp.minimum(rn, quant - 1).astype(jnp.int32)
    return _run(output1, output2, rn)


# speedup vs baseline: 1.0313x; 1.0207x over previous
import functools

import jax
import jax.numpy as jnp
from jax import lax
from jax.experimental import pallas as pl
from jax.experimental.pallas import tpu as pltpu, tpu_sc as plsc

_N = 4096
_D = 512
_BLK = 256
_NC = 2
_NS = 16
_NW = _NC * _NS
_RPW = _N // _NW


def _make_tc_body(off):
    def _tc_body(o1_ref, o2t_ref, keys_ref, pos_ref):
        return _tc_body_impl(off, o1_ref, o2t_ref, keys_ref, pos_ref)
    return _tc_body


def _tc_body_impl(off, o1_ref, o2t_ref, keys_ref, pos_ref):
    r0 = off + pl.program_id(0) * _BLK
    o1 = o1_ref[...]
    o2t = o2t_ref[...]
    n1 = jnp.sum(o1 * o1, axis=1, keepdims=True)
    n2 = jnp.sum(o2t * o2t, axis=0, keepdims=True)
    dot = jnp.dot(o1.astype(jnp.bfloat16), o2t.astype(jnp.bfloat16),
                  preferred_element_type=jnp.float32)
    d2 = n1 - 2.0 * dot + n2
    cols = jax.lax.broadcasted_iota(jnp.int32, (_BLK, _N), 1)
    rows = jax.lax.broadcasted_iota(jnp.int32, (_BLK, _N), 0) + r0
    diag = cols == rows
    d2 = jnp.where(diag, jnp.inf, d2)
    d2 = jnp.maximum(d2, 1e-12)
    keys_ref[...] = jax.lax.bitcast_convert_type(d2, jnp.int32)

    dmask = diag.astype(jnp.float32)
    dd = jnp.sum(dot * dmask, axis=1, keepdims=True)
    n2d = jnp.sum(n2 * dmask, axis=1, keepdims=True)
    pos_ref[...] = n1 + n2d - 2.0 * dd


def _splat(x):
    return jnp.full((16,), x, jnp.int32)


_GDN = jax.lax.GatherDimensionNumbers(
    offset_dims=(), collapsed_slice_dims=(0,), start_index_map=(0,))


def _lane_gather(x, idx_vec):
    return jax.lax.gather(x, idx_vec[:, None], _GDN, (1,),
                          mode=jax.lax.GatherScatterMode.PROMISE_IN_BOUNDS)


def _last_lane(x):
    return _lane_gather(x, jnp.full((16,), 15, jnp.int32))


def _sc_body(keys_hbm, rn_hbm, out_hbm, kbuf, hist, hist2, coarse, coarse2,
             rnv, chosen, sems):
    nrows = rn_hbm.shape[0]
    rpw = nrows // _NW
    wid = lax.axis_index("s") * _NC + lax.axis_index("c")
    base = wid * rpw
    pltpu.sync_copy(rn_hbm.at[pl.ds(base, rpw)], rnv)

    lanes = lax.iota(jnp.int32, 16)
    zeros16 = jnp.zeros((16,), jnp.int32)
    ones16 = jnp.full((16,), 1, jnp.int32)
    lane0 = lanes == 0

    def _block_copy(rb, par):
        return pltpu.make_async_copy(
            keys_hbm.at[pl.ds(base + rb * 8, 8)], kbuf.at[par], sems.at[par])

    _block_copy(0, 0).start()

    def block_step(rb, _):
        par = rb % 2
        _block_copy(rb, par).wait()

        @pl.when(rb < rpw // 8 - 1)
        def _start_next():
            _block_copy(rb + 1, 1 - par).start()

        def row_step(rloc, _):
            r = rb * 8 + rloc
            _select_row(r, rloc, par)
            return 0
        lax.fori_loop(0, 8, row_step, 0)
        return 0

    def _select_row(r, rloc, par):
        @plsc.parallel_loop(0, 2048, step=16, unroll=8)
        def _z1(i):
            hist[pl.ds(i, 16)] = zeros16

        @plsc.parallel_loop(0, 1024, step=16, unroll=8)
        def _z2(i):
            hist2[pl.ds(i, 16)] = zeros16

        for t in range(8):
            coarse[pl.ds(t * 16, 16)] = zeros16
        for t in range(4):
            coarse2[pl.ds(t * 16, 16)] = zeros16

        k_vec = plsc.load_gather(rnv, [_splat(r)])

        @plsc.parallel_loop(0, _N, step=16, unroll=8)
        def _p1(i):
            v = kbuf[par, rloc, pl.ds(i, 16)]
            b = (v >> 20) & 2047
            plsc.addupdate_scatter(hist, [b], ones16)
            cc, last = plsc.scan_count(b >> 4)
            plsc.addupdate_scatter(coarse, [b >> 4], cc, mask=last)

        x1, rin1 = _scan_find(coarse, 8, k_vec)
        b1, r2 = _fine_find(hist, x1, rin1)

        @plsc.parallel_loop(0, _N, step=16, unroll=8)
        def _p2(i):
            v = kbuf[par, rloc, pl.ds(i, 16)]
            t = v >> 10
            m = (t >> 10) == b1
            b = t & 1023
            plsc.addupdate_scatter(hist2, [b], ones16, mask=m)
            cc, last = plsc.scan_count(b >> 4, mask=m)
            plsc.addupdate_scatter(coarse2, [b >> 4], cc, mask=last & m)

        x2, rin2 = _scan_find(coarse2, 4, r2)
        b2, _ = _fine_find(hist2, x2, rin2)

        bits = (b1 << 20) | (b2 << 10) | 512
        plsc.store_scatter(chosen, [_splat(r)], bits, mask=lane0)

    def _scan_find(c_ref, nt, rank):
        cs = [plsc.cumsum(c_ref[pl.ds(16 * t, 16)]) for t in range(nt)]
        run = zeros16
        cnt = zeros16
        below_acc = zeros16
        for t in range(nt):
            c = run + cs[t]
            m = c <= rank
            cnt = cnt + plsc.all_reduce_population_count(m)
            below_acc = jnp.maximum(below_acc, jnp.where(m, c, 0))
            if t < nt - 1:
                run = _last_lane(c)
        below = _splat(jnp.max(below_acc))
        return cnt, rank - below

    def _fine_find(h_ref, x, rank):
        h = plsc.load_gather(h_ref, [x * 16 + lanes])
        c = plsc.cumsum(h)
        m = c <= rank
        lane = plsc.all_reduce_population_count(m)
        below = jnp.where(lane == 0, 0,
                          _lane_gather(c, jnp.maximum(lane - 1, 0)))
        return x * 16 + lane, rank - below

    lax.fori_loop(0, rpw // 8, block_step, 0)
    pltpu.sync_copy(chosen, out_hbm.at[pl.ds(base, rpw)])


_NH = 2
_HR = _N // _NH


@jax.jit
def _run(output1, output2, rn):
    o2t = output2.T

    def tc_half(h):
        return pl.pallas_call(
            _make_tc_body(h * _HR),
            grid=(_HR // _BLK,),
            in_specs=[
                pl.BlockSpec((_BLK, _D), lambda i: (i, 0)),
                pl.BlockSpec((_D, _N), lambda i: (0, 0)),
            ],
            out_specs=[
                pl.BlockSpec((_BLK, _N), lambda i: (i, 0)),
                pl.BlockSpec((_BLK, 1), lambda i: (i, 0)),
            ],
            out_shape=[
                jax.ShapeDtypeStruct((_HR, _N), jnp.int32),
                jax.ShapeDtypeStruct((_HR, 1), jnp.float32),
            ],
        )(output1[h * _HR:(h + 1) * _HR], o2t)

    rpw = _HR // _NW
    sc_select = pl.kernel(
        _sc_body,
        out_type=jax.ShapeDtypeStruct((_HR,), jnp.int32),
        mesh=plsc.VectorSubcoreMesh(core_axis_name="c", subcore_axis_name="s"),
        compiler_params=pltpu.CompilerParams(needs_layout_passes=False),
        scratch_types=[
            pltpu.VMEM((2, 8, _N), jnp.int32),
            pltpu.VMEM((2048,), jnp.int32),
            pltpu.VMEM((1024,), jnp.int32),
            pltpu.VMEM((128,), jnp.int32),
            pltpu.VMEM((64,), jnp.int32),
            pltpu.VMEM((rpw,), jnp.int32),
            pltpu.VMEM((rpw,), jnp.int32),
            pltpu.SemaphoreType.DMA((2,)),
        ],
    )

    chosen_parts = []
    pos_parts = []
    for h in range(_NH):
        keys_h, pos_h = tc_half(h)
        chosen_parts.append(sc_select(keys_h, rn[h * _HR:(h + 1) * _HR]))
        pos_parts.append(pos_h)

    chosen_bits = jnp.concatenate(chosen_parts)
    pos = jnp.concatenate(pos_parts)
    dist = jnp.sqrt(jax.lax.bitcast_convert_type(chosen_bits, jnp.float32))
    neg_loss = jnp.clip(2.0 - dist, 0.0, None)
    return jnp.mean(pos[:, 0]) + jnp.mean(neg_loss)


def kernel(output1, output2, quant):
    N = output1.shape[0]
    q = min(100, N - 1)
    rn = jax.random.randint(jax.random.key(1234), (N,), 0, q)
    rn = jnp.minimum(rn, quant - 1).astype(jnp.int32)
    return _run(output1, output2, rn)
